# baseline (speedup 1.0000x reference)
import jax
import jax.numpy as jnp
from jax import lax
from jax.experimental import pallas as pl
from jax.experimental.pallas import tpu as pltpu
from jax.experimental.pallas import tpu_sc as plsc

MAX_LEN = 8192
DIM = 1024
ROWS_PER_CORE = MAX_LEN // 2
CHUNK = 512
CHUNKS_PER_CORE = ROWS_PER_CORE // CHUNK
NBUF = 3

_mesh = plsc.VectorSubcoreMesh(core_axis_name="c", subcore_axis_name="s")


def _sc_body(t_hbm, pe_hbm, zeros_hbm, out_hbm,
             t_v, spmem, si0, si1, si2, so0, so1, so2):
    sid = lax.axis_index("s")
    cid = lax.axis_index("c")
    base = cid * ROWS_PER_CORE

    sin = (si0, si1, si2)
    sout = (so0, so1, so2)

    @pl.when(sid == 0)
    def _active():
        pltpu.sync_copy(t_hbm, t_v)
        t = t_v[...][0]
        bufs = tuple(spmem.at[b] for b in range(NBUF))

        @pl.when(base + ROWS_PER_CORE <= t)
        def _fast_copy():
            n = CHUNKS_PER_CORE
            h_in = [None] * n
            h_out = [None] * n
            for i in range(n):
                b = i % NBUF
                if i >= NBUF:
                    h_out[i - NBUF].wait()
                src = pe_hbm.at[pl.ds(base + i * CHUNK, CHUNK)]
                h_in[i] = pltpu.async_copy(src, bufs[b], sin[b])
                if i >= 1:
                    h_in[i - 1].wait()
                    pb = (i - 1) % NBUF
                    dst = out_hbm.at[pl.ds(base + (i - 1) * CHUNK, CHUNK)]
                    h_out[i - 1] = pltpu.async_copy(bufs[pb], dst, sout[pb])
            h_in[n - 1].wait()
            lb = (n - 1) % NBUF
            dst = out_hbm.at[pl.ds(base + (n - 1) * CHUNK, CHUNK)]
            h_out[n - 1] = pltpu.async_copy(bufs[lb], dst, sout[lb])
            for i in range(n - NBUF, n):
                if i >= 0:
                    h_out[i].wait()

        @pl.when(base + ROWS_PER_CORE > t)
        def _masked_path():
            buf0 = bufs[0]
            for ci in range(CHUNKS_PER_CORE):
                cbase = base + ci * CHUNK

                @pl.when(cbase + CHUNK <= t)
                def _copy_chunk():
                    pltpu.sync_copy(pe_hbm.at[pl.ds(cbase, CHUNK)], buf0)
                    pltpu.sync_copy(buf0, out_hbm.at[pl.ds(cbase, CHUNK)])

                @pl.when(cbase >= t)
                def _zero_chunk():
                    pltpu.sync_copy(zeros_hbm, buf0)
                    pltpu.sync_copy(buf0, out_hbm.at[pl.ds(cbase, CHUNK)])

                @pl.when(jnp.logical_and(cbase < t, cbase + CHUNK > t))
                def _straddle_chunk():
                    pltpu.sync_copy(pe_hbm.at[pl.ds(cbase, CHUNK)], buf0)

                    def row_body(r, carry):
                        @pl.when(cbase + r >= t)
                        def _zero_row():
                            pltpu.sync_copy(zeros_hbm.at[0], buf0.at[r])

                        return carry

                    lax.fori_loop(0, CHUNK, row_body, 0)
                    pltpu.sync_copy(buf0, out_hbm.at[pl.ds(cbase, CHUNK)])


_sc_call = pl.kernel(
    _sc_body,
    mesh=_mesh,
    out_type=jax.ShapeDtypeStruct((MAX_LEN, DIM), jnp.float32),
    scratch_types=(
        [pltpu.VMEM((16,), jnp.int32),
         pltpu.VMEM_SHARED((NBUF, CHUNK, DIM), jnp.float32)]
        + [pltpu.SemaphoreType.DMA] * (2 * NBUF)
    ),
)


def kernel(pe, T):
    t_arr = jnp.full((16,), T, dtype=jnp.int32)
    zeros = jnp.zeros((CHUNK, DIM), dtype=jnp.float32)
    out = _sc_call(t_arr, pe, zeros)
    return out[None, :, :]


# speedup vs baseline: 2.1711x; 2.1711x over previous
import jax
import jax.numpy as jnp
from jax.experimental import pallas as pl
from jax.experimental.pallas import tpu as pltpu

MAX_LEN = 8192
DIM = 1024
BLOCK_ROWS = 2048


def _body(t_ref, pe_ref, out_ref):
    i = pl.program_id(0)
    t = t_ref[0]
    blk_start = i * BLOCK_ROWS

    @pl.when(blk_start + BLOCK_ROWS <= t)
    def _full_copy():
        out_ref[...] = pe_ref[...]

    @pl.when(blk_start + BLOCK_ROWS > t)
    def _masked_copy():
        rows = jax.lax.broadcasted_iota(jnp.int32, (BLOCK_ROWS, 1), 0) + blk_start
        out_ref[...] = jnp.where(rows < t, pe_ref[...], 0.0)


def kernel(pe, T):
    t_arr = jnp.asarray(T, dtype=jnp.int32).reshape((1,))
    n_blocks = MAX_LEN // BLOCK_ROWS
    out = pl.pallas_call(
        _body,
        grid=(n_blocks,),
        in_specs=[
            pl.BlockSpec(memory_space=pltpu.SMEM),
            pl.BlockSpec((BLOCK_ROWS, DIM), lambda i: (i, 0)),
        ],
        out_specs=pl.BlockSpec((BLOCK_ROWS, DIM), lambda i: (i, 0)),
        out_shape=jax.ShapeDtypeStruct((MAX_LEN, DIM), jnp.float32),
    )(t_arr, pe)
    return out[None, :, :]
